# baseline (speedup 1.0000x reference)
import functools

import jax
import jax.numpy as jnp
from jax import lax
from jax.experimental import pallas as pl
from jax.experimental.pallas import tpu as pltpu
from jax.experimental.pallas import tpu_sc as plsc

B, N = 8, 65536
H, W = 512, 512
NC, NS, L = 2, 16, 16
NW = NC * NS
P = B * N
PPT = P // NW
CHUNK = 2048
GSZ = 128
NSTREAM = CHUNK // GSZ


def _mask_body(coords_hbm, terrain_hbm, out_hbm, cbuf, ibuf, zbuf, gbuf,
               obuf, gsem):
    c = lax.axis_index("c")
    s = lax.axis_index("s")
    wid = s * NC + c
    base = wid * PPT
    batch = wid // (NW // B)
    bofs = batch * (H * W)
    lanes = lax.iota(jnp.int32, L)
    stride4 = lanes * 4

    def chunk_body(k, carry):
        cb = base + k * CHUNK
        pltpu.sync_copy(coords_hbm.at[pl.ds(cb * 4, CHUNK * 4)], cbuf)

        def vec_body(i, carry2):
            off = i * (L * 4) + stride4
            xv = plsc.load_gather(cbuf, [off])
            yv = plsc.load_gather(cbuf, [off + 1])
            zv = plsc.load_gather(cbuf, [off + 2])
            xq = jnp.clip((xv * 511.0).astype(jnp.int32), 0, W - 1)
            yq = jnp.clip((yv * 511.0).astype(jnp.int32), 0, H - 1)
            ibuf[pl.ds(i * L, L)] = bofs + yq * W + xq
            zbuf[pl.ds(i * L, L)] = zv
            return carry2

        lax.fori_loop(0, CHUNK // L, vec_body, 0, unroll=4)

        def fire(j, carry2):
            pltpu.make_async_copy(
                terrain_hbm.at[ibuf.at[pl.ds(j * GSZ, GSZ)]],
                gbuf.at[pl.ds(j * GSZ, GSZ)],
                gsem,
            ).start()
            return carry2

        lax.fori_loop(0, NSTREAM, fire, 0)

        def drain(j, carry2):
            pltpu.make_async_copy(
                terrain_hbm.at[ibuf.at[pl.ds(j * GSZ, GSZ)]],
                gbuf.at[pl.ds(j * GSZ, GSZ)],
                gsem,
            ).wait()
            return carry2

        lax.fori_loop(0, NSTREAM, drain, 0)

        def cmp_body(i, carry2):
            tv = gbuf[pl.ds(i * L, L)]
            zv = zbuf[pl.ds(i * L, L)]
            obuf[pl.ds(i * L, L)] = jnp.where(zv <= tv, 1.0, 0.0)
            return carry2

        lax.fori_loop(0, CHUNK // L, cmp_body, 0, unroll=4)

        pltpu.sync_copy(obuf, out_hbm.at[pl.ds(cb, CHUNK)])
        return carry

    lax.fori_loop(0, PPT // CHUNK, chunk_body, 0)


@jax.jit
def _launch(coords_flat, terrain_flat):
    mesh = plsc.VectorSubcoreMesh(core_axis_name="c", subcore_axis_name="s")
    kern = functools.partial(
        pl.kernel,
        mesh=mesh,
        out_type=jax.ShapeDtypeStruct((P,), jnp.float32),
        scratch_types=[
            pltpu.VMEM((CHUNK * 4,), jnp.float32),
            pltpu.VMEM((CHUNK,), jnp.int32),
            pltpu.VMEM((CHUNK,), jnp.float32),
            pltpu.VMEM((CHUNK,), jnp.float32),
            pltpu.VMEM((CHUNK,), jnp.float32),
            pltpu.SemaphoreType.DMA,
        ],
        compiler_params=pltpu.CompilerParams(needs_layout_passes=False),
    )(_mask_body)
    return kern(coords_flat, terrain_flat)


def kernel(coords, terrain_height):
    coords_flat = coords.reshape(-1)
    terrain_flat = terrain_height.reshape(-1)
    out = _launch(coords_flat, terrain_flat)
    return out.reshape(B, N, 1)


# speedup vs baseline: 11.1532x; 11.1532x over previous
import functools

import jax
import jax.numpy as jnp
from jax import lax
from jax.experimental import pallas as pl
from jax.experimental.pallas import tpu as pltpu
from jax.experimental.pallas import tpu_sc as plsc

B, N = 8, 65536
H, W = 512, 512
NC, NS, L = 2, 16, 16
NW = NC * NS
P = B * N
PPT = P // NW
CHUNK = 2048
GSZ = 128
NSTREAM = CHUNK // GSZ


def _mask_body(coords_hbm, terrain_hbm, out_hbm, cbuf, ibuf, gbuf, obuf,
               gsem):
    c = lax.axis_index("c")
    s = lax.axis_index("s")
    wid = s * NC + c
    base = wid * PPT
    batch = wid // (NW // B)
    bofs = batch * (H * W)

    def chunk_body(k, carry):
        cb = base + k * CHUNK
        pltpu.sync_copy(coords_hbm.at[pl.ds(cb * 4, CHUNK * 4)], cbuf)

        def vec_body(i, carry2):
            g = i // 8
            r = i % 8
            off = g * 512 + r * 16
            xv = cbuf[pl.ds(off, L)]
            yv = cbuf[pl.ds(off + 128, L)]
            xq = jnp.clip((xv * 511.0).astype(jnp.int32), 0, W - 1)
            yq = jnp.clip((yv * 511.0).astype(jnp.int32), 0, H - 1)
            pidx = (bofs + (yq >> 3) * 3072 + (yq << 7)
                    + (xq >> 7) * 896 + xq)
            ibuf[pl.ds(i * L, L)] = pidx
            return carry2

        lax.fori_loop(0, CHUNK // L, vec_body, 0, unroll=8)

        def fire(j, carry2):
            pltpu.make_async_copy(
                terrain_hbm.at[ibuf.at[pl.ds(j * GSZ, GSZ)]],
                gbuf.at[pl.ds(j * GSZ, GSZ)],
                gsem,
            ).start()
            return carry2

        lax.fori_loop(0, NSTREAM, fire, 0)

        def drain(j, carry2):
            pltpu.make_async_copy(
                terrain_hbm.at[ibuf.at[pl.ds(j * GSZ, GSZ)]],
                gbuf.at[pl.ds(j * GSZ, GSZ)],
                gsem,
            ).wait()
            return carry2

        lax.fori_loop(0, NSTREAM, drain, 0)

        def cmp_body(i, carry2):
            g = i // 8
            r = i % 8
            zv = cbuf[pl.ds(g * 512 + 256 + r * 16, L)]
            tv = gbuf[pl.ds(i * L, L)]
            obuf[pl.ds(i * L, L)] = jnp.where(zv <= tv, 1.0, 0.0)
            return carry2

        lax.fori_loop(0, CHUNK // L, cmp_body, 0, unroll=8)

        pltpu.sync_copy(obuf, out_hbm.at[pl.ds(cb, CHUNK)])
        return carry

    lax.fori_loop(0, PPT // CHUNK, chunk_body, 0)


@jax.jit
def _launch(coords_phys, terrain_phys):
    mesh = plsc.VectorSubcoreMesh(core_axis_name="c", subcore_axis_name="s")
    kern = functools.partial(
        pl.kernel,
        mesh=mesh,
        out_type=jax.ShapeDtypeStruct((P,), jnp.float32),
        scratch_types=[
            pltpu.VMEM((CHUNK * 4,), jnp.float32),
            pltpu.VMEM((CHUNK,), jnp.int32),
            pltpu.VMEM((CHUNK,), jnp.float32),
            pltpu.VMEM((CHUNK,), jnp.float32),
            pltpu.SemaphoreType.DMA,
        ],
        compiler_params=pltpu.CompilerParams(needs_layout_passes=False),
    )(_mask_body)
    return kern(coords_phys, terrain_phys)


def kernel(coords, terrain_height):
    coords_phys = (
        coords.reshape(B, N // 128, 128, 4)
        .transpose(0, 1, 3, 2)
        .reshape(-1)
    )
    terrain_phys = (
        terrain_height.reshape(B, H // 8, 8, W // 128, 128)
        .transpose(0, 1, 3, 2, 4)
        .reshape(-1)
    )
    out = _launch(coords_phys, terrain_phys)
    return out.reshape(B, N, 1)


# speedup vs baseline: 14.4858x; 1.2988x over previous
import functools

import jax
import jax.numpy as jnp
from jax import lax
from jax.experimental import pallas as pl
from jax.experimental.pallas import tpu as pltpu
from jax.experimental.pallas import tpu_sc as plsc

B, N = 8, 65536
H, W = 512, 512
NC, NS, L = 2, 16, 16
NW = NC * NS
P = B * N
PPT = P // NW
CHUNK = 2048
K = PPT // CHUNK
GSZ = 128
NSTREAM = CHUNK // GSZ
GROUPS = CHUNK // 128


def _mask_body(coords_hbm, terrain_hbm, out_hbm,
               cbuf0, cbuf1, ibuf0, ibuf1, gbuf0, gbuf1, obuf0, obuf1,
               csem0, csem1, gsem0, gsem1, osem0, osem1):
    c = lax.axis_index("c")
    s = lax.axis_index("s")
    wid = s * NC + c
    base = wid * PPT
    batch = wid // (NW // B)
    bofs = batch * (H * W)

    cbufs, ibufs = (cbuf0, cbuf1), (ibuf0, ibuf1)
    gbufs, obufs = (gbuf0, gbuf1), (obuf0, obuf1)
    csems, gsems, osems = (csem0, csem1), (gsem0, gsem1), (osem0, osem1)

    def coords_dma(k):
        cb = base + k * CHUNK
        return pltpu.make_async_copy(
            coords_hbm.at[pl.ds(cb * 4, CHUNK * 4)], cbufs[k % 2],
            csems[k % 2])

    def out_dma(k):
        cb = base + k * CHUNK
        return pltpu.make_async_copy(
            obufs[k % 2], out_hbm.at[pl.ds(cb, CHUNK)], osems[k % 2])

    def gather(k, j):
        return pltpu.make_async_copy(
            terrain_hbm.at[ibufs[k % 2].at[pl.ds(j * GSZ, GSZ)]],
            gbufs[k % 2].at[pl.ds(j * GSZ, GSZ)], gsems[k % 2])

    def compute_idx(k):
        cbuf, ibuf = cbufs[k % 2], ibufs[k % 2]

        def body(g, carry):
            for r in range(8):
                off = g * 512 + r * 16
                xv = cbuf[pl.ds(off, L)]
                yv = cbuf[pl.ds(off + 128, L)]
                xq = (xv * 511.0).astype(jnp.int32)
                yq = (yv * 511.0).astype(jnp.int32)
                pidx = (bofs + (yq >> 3) * 3072 + (yq << 7)
                        + (xq >> 7) * 896 + xq)
                ibuf[pl.ds(g * 128 + r * 16, L)] = pidx
            return carry

        lax.fori_loop(0, GROUPS, body, 0)

    def compare(k):
        cbuf, gbuf, obuf = cbufs[k % 2], gbufs[k % 2], obufs[k % 2]

        def body(g, carry):
            for r in range(8):
                zv = cbuf[pl.ds(g * 512 + 256 + r * 16, L)]
                tv = gbuf[pl.ds(g * 128 + r * 16, L)]
                obuf[pl.ds(g * 128 + r * 16, L)] = jnp.where(
                    zv <= tv, 1.0, 0.0)
            return carry

        lax.fori_loop(0, GROUPS, body, 0)

    coords_dma(0).start()
    coords_dma(1).start()
    coords_dma(0).wait()
    compute_idx(0)
    for j in range(NSTREAM):
        gather(0, j).start()

    for k in range(K):
        if k + 1 < K:
            coords_dma(k + 1).wait()
            compute_idx(k + 1)
        for j in range(NSTREAM):
            gather(k, j).wait()
        if k + 1 < K:
            for j in range(NSTREAM):
                gather(k + 1, j).start()
        if k >= 2:
            out_dma(k - 2).wait()
        compare(k)
        out_dma(k).start()
        if k + 2 < K:
            coords_dma(k + 2).start()

    out_dma(K - 2).wait()
    out_dma(K - 1).wait()


@jax.jit
def _launch(coords_phys, terrain_phys):
    mesh = plsc.VectorSubcoreMesh(core_axis_name="c", subcore_axis_name="s")
    kern = functools.partial(
        pl.kernel,
        mesh=mesh,
        out_type=jax.ShapeDtypeStruct((P,), jnp.float32),
        scratch_types=[
            pltpu.VMEM((CHUNK * 4,), jnp.float32),
            pltpu.VMEM((CHUNK * 4,), jnp.float32),
            pltpu.VMEM((CHUNK,), jnp.int32),
            pltpu.VMEM((CHUNK,), jnp.int32),
            pltpu.VMEM((CHUNK,), jnp.float32),
            pltpu.VMEM((CHUNK,), jnp.float32),
            pltpu.VMEM((CHUNK,), jnp.float32),
            pltpu.VMEM((CHUNK,), jnp.float32),
            pltpu.SemaphoreType.DMA,
            pltpu.SemaphoreType.DMA,
            pltpu.SemaphoreType.DMA,
            pltpu.SemaphoreType.DMA,
            pltpu.SemaphoreType.DMA,
            pltpu.SemaphoreType.DMA,
        ],
        compiler_params=pltpu.CompilerParams(needs_layout_passes=False),
    )(_mask_body)
    return kern(coords_phys, terrain_phys)


def kernel(coords, terrain_height):
    coords_phys = (
        coords.reshape(B, N // 128, 128, 4)
        .transpose(0, 1, 3, 2)
        .reshape(-1)
    )
    terrain_phys = (
        terrain_height.reshape(B, H // 8, 8, W // 128, 128)
        .transpose(0, 1, 3, 2, 4)
        .reshape(-1)
    )
    out = _launch(coords_phys, terrain_phys)
    return out.reshape(B, N, 1)
